# baseline (speedup 1.0000x reference)
import jax
import jax.numpy as jnp
from jax.experimental import pallas as pl

_S = 16
_V = 431
_C = 1280
_D = 8

_LNPW, _LNPB = 0, 16
_L1W, _L1B = 32, 160
_LN1W, _LN1B = 168, 176
_GW, _GB = 184, 248
_LN2W, _LN2B = 256, 264
_L2W, _L2B = 272, 400
_NP = 416


def _body(h_ref, w1_ref, b1_ref, adj_ref, w3_ref, b3_ref, p_ref, out_ref):
    f32 = jnp.float32
    bf16 = jnp.bfloat16
    n = h_ref.shape[0]

    def prow(r):
        return p_ref[r:r + 1, :]

    ck = 128
    nk = _C // ck
    pieces = [[] for _ in range(_S)]
    for k in range(nk):
        hc = h_ref[:, k * ck * _S:(k + 1) * ck * _S].astype(bf16)
        hct = hc.T.reshape(ck, _S, n)
        for s in range(_S):
            pieces[s].append(hct[:, s, :])
    hs = [jnp.concatenate(pieces[s], axis=0) for s in range(_S)]

    w1 = w1_ref[...]
    b1 = b1_ref[...]
    x = [jnp.dot(w1, hs[s], preferred_element_type=f32) + b1
         for s in range(_S)]

    u = x[0]
    for s in range(1, _S):
        u = u + x[s]
    u = u * (1.0 / _S)
    d = [x[s] - u for s in range(_S)]
    var = d[0] * d[0]
    for s in range(1, _S):
        var = var + d[s] * d[s]
    r = jax.lax.rsqrt(var * (1.0 / _S) + 1e-12)
    t = [jnp.maximum(d[s] * r * prow(_LNPW + s) + prow(_LNPB + s), 0.0)
         for s in range(_S)]

    y = []
    for dd in range(_D):
        acc = t[0] * prow(_L1W + dd * _S)
        for s in range(1, _S):
            acc = acc + t[s] * prow(_L1W + dd * _S + s)
        y.append(acc + prow(_L1B + dd))

    u = y[0]
    for dd in range(1, _D):
        u = u + y[dd]
    u = u * (1.0 / _D)
    d = [y[dd] - u for dd in range(_D)]
    var = d[0] * d[0]
    for dd in range(1, _D):
        var = var + d[dd] * d[dd]
    r = jax.lax.rsqrt(var * (1.0 / _D) + 1e-12)
    y = [jnp.maximum(d[dd] * r * prow(_LN1W + dd) + prow(_LN1B + dd), 0.0)
         for dd in range(_D)]

    adj = adj_ref[...]
    for _ in range(2):
        g = []
        for d2 in range(_D):
            acc = y[0] * prow(_GW + d2)
            for d1 in range(1, _D):
                acc = acc + y[d1] * prow(_GW + d1 * _D + d2)
            g.append(acc.astype(bf16))
        y_all = jnp.dot(adj, jnp.concatenate(g, axis=1),
                        preferred_element_type=f32)
        y = [y_all[:, d2 * n:(d2 + 1) * n] + prow(_GB + d2)
             for d2 in range(_D)]

    u = y[0]
    for dd in range(1, _D):
        u = u + y[dd]
    u = u * (1.0 / _D)
    d = [y[dd] - u for dd in range(_D)]
    var = d[0] * d[0]
    for dd in range(1, _D):
        var = var + d[dd] * d[dd]
    r = jax.lax.rsqrt(var * (1.0 / _D) + 1e-12)
    t2 = [jnp.maximum(d[dd] * r * prow(_LN2W + dd) + prow(_LN2B + dd), 0.0)
          for dd in range(_D)]

    z = []
    for s in range(_S):
        acc = t2[0] * prow(_L2W + s * _D)
        for dd in range(1, _D):
            acc = acc + t2[dd] * prow(_L2W + s * _D + dd)
        z.append((x[s] + acc + prow(_L2B + s)).astype(bf16))
    z_all = jnp.concatenate(z, axis=1)

    for k in range(nk):
        o_k = (jnp.dot(w3_ref[k * ck:(k + 1) * ck, :], z_all,
                       preferred_element_type=f32)
               + b3_ref[k * ck:(k + 1) * ck, :])
        ot_k = jnp.stack([o_k[:, s * n:(s + 1) * n] for s in range(_S)],
                         axis=1)
        out_ref[:, k * ck * _S:(k + 1) * ck * _S] = ot_k.reshape(ck * _S, n).T


def kernel(hidden_states, W1, b1, ln_pre_w, ln_pre_b, lin1_w, lin1_b,
           ln1_w, ln1_b, gcn_w, gcn_b, adjmat, ln2_w, ln2_b,
           lin2_w, lin2_b, W3, b3):
    T = hidden_states.shape[2]
    hp = hidden_states.reshape(-1, _C * _S)
    n = hp.shape[0]

    rows = jnp.concatenate([
        ln_pre_w, ln_pre_b,
        lin1_w.reshape(-1), lin1_b,
        ln1_w, ln1_b,
        gcn_w.reshape(-1), gcn_b,
        ln2_w, ln2_b,
        lin2_w.reshape(-1), lin2_b,
    ]).astype(jnp.float32)
    params = jnp.broadcast_to(rows[:, None], (_NP, n))

    out = pl.pallas_call(
        _body,
        out_shape=jax.ShapeDtypeStruct((n, _C * _S), jnp.float32),
    )(hp, W1.astype(jnp.bfloat16), b1.reshape(_V, 1),
      adjmat.astype(jnp.bfloat16), W3.astype(jnp.bfloat16),
      b3.reshape(_C, 1), params)

    return out.reshape(-1, _C, T, 4, 4)
